# Initial kernel scaffold; baseline (speedup 1.0000x reference)
import jax
import jax.numpy as jnp
from jax.experimental import pallas as pl


def kernel(constraint_features, edge_indices, edge_features, variable_features, params):
    raise NotImplementedError("write your pallas kernel here")


# speedup vs baseline: 1.4452x; 1.4452x over previous
import functools

import jax
import jax.numpy as jnp
from jax import lax
from jax.experimental import pallas as pl
from jax.experimental.pallas import tpu as pltpu
from jax.experimental.pallas import tpu_sc as plsc

EMB = 64
HALF = 32
N_NODES = 50000
N_EDGES = 800000

NC = 2
NS = 16

SHARE = N_EDGES // NS
CH = 80
NSUB = 5
SCH = CH * NSUB
NSUP = SHARE // SCH

NPAD = 50048
ZROWS = 391
WB = N_NODES // NS

_mesh = plsc.VectorSubcoreMesh(
    core_axis_name="c", subcore_axis_name="s", num_cores=NC, num_subcores=NS)

f32 = jnp.float32


BN = 2000
BE = 3200


def _row_spec(bn, d):
  return pl.BlockSpec((bn, d), lambda i: (i, 0))


def _full_spec(shape):
  return pl.BlockSpec(shape, lambda i: tuple(0 for _ in shape))


def _ln(x, g, b):
  m = jnp.mean(x, axis=-1, keepdims=True)
  v = jnp.mean((x - m) * (x - m), axis=-1, keepdims=True)
  return (x - m) * lax.rsqrt(v + 1e-5) * g + b


def _embed_body(x_ref, g_ref, b_ref, w1_ref, b1_ref, w2_ref, b2_ref, o_ref):
  x = _ln(x_ref[...], g_ref[...], b_ref[...])
  h = jnp.maximum(jnp.dot(x, w1_ref[...], preferred_element_type=f32, precision=lax.Precision.HIGHEST)
                  + b1_ref[...], 0.0)
  o_ref[...] = jnp.maximum(jnp.dot(h, w2_ref[...], preferred_element_type=f32, precision=lax.Precision.HIGHEST)
                           + b2_ref[...], 0.0)


def _embed(x, g, b, w1, b1, w2, b2):
  n, d = x.shape
  return pl.pallas_call(
      _embed_body,
      grid=(n // BN,),
      in_specs=[_row_spec(BN, d), _full_spec((1, d)), _full_spec((1, d)),
                _full_spec((d, EMB)), _full_spec((1, EMB)),
                _full_spec((EMB, EMB)), _full_spec((1, EMB))],
      out_specs=_row_spec(BN, EMB),
      out_shape=jax.ShapeDtypeStruct((n, EMB), f32),
  )(x, g.reshape(1, d), b.reshape(1, d), w1, b1.reshape(1, EMB),
    w2, b2.reshape(1, EMB))


def _ab_body(r_ref, l_ref, flw_ref, fla_ref, frw_ref,
             al_ref, ar_ref, bl_ref, br_ref):
  a = jnp.dot(r_ref[...], flw_ref[...], preferred_element_type=f32, precision=lax.Precision.HIGHEST) + fla_ref[...]
  bb = jnp.dot(l_ref[...], frw_ref[...], preferred_element_type=f32, precision=lax.Precision.HIGHEST)
  al_ref[...] = a[:, :HALF]
  ar_ref[...] = a[:, HALF:]
  bl_ref[...] = bb[:, :HALF]
  br_ref[...] = bb[:, HALF:]


def _ab(right, left, fl_w, bias_a, fr_w):
  n = right.shape[0]
  sds = jax.ShapeDtypeStruct((n, HALF), f32)
  return pl.pallas_call(
      _ab_body,
      grid=(n // BN,),
      in_specs=[_row_spec(BN, EMB), _row_spec(BN, EMB),
                _full_spec((EMB, EMB)), _full_spec((1, EMB)),
                _full_spec((EMB, EMB))],
      out_specs=[_row_spec(BN, HALF)] * 4,
      out_shape=[sds] * 4,
  )(right, left, fl_w, bias_a.reshape(1, EMB), fr_w)


def _edge_body(sl_ref, sr_ref, g_ref, be_ref, w_ref, b_ref, ml_ref, mr_ref):
  h = jnp.concatenate([sl_ref[...], sr_ref[...]], axis=-1)
  h = jnp.maximum(_ln(h, g_ref[...], be_ref[...]), 0.0)
  msg = jnp.dot(h, w_ref[...], preferred_element_type=f32, precision=lax.Precision.HIGHEST) + b_ref[...]
  ml_ref[...] = msg[:, :HALF]
  mr_ref[...] = msg[:, HALF:]


def _edge_mlp(sl, sr, g, beta, w, b):
  e = sl.shape[0]
  sds = jax.ShapeDtypeStruct((e, HALF), f32)
  return pl.pallas_call(
      _edge_body,
      grid=(e // BE,),
      in_specs=[_row_spec(BE, HALF), _row_spec(BE, HALF),
                _full_spec((1, EMB)), _full_spec((1, EMB)),
                _full_spec((EMB, EMB)), _full_spec((1, EMB))],
      out_specs=[_row_spec(BE, HALF)] * 2,
      out_shape=[sds] * 2,
  )(sl, sr, g.reshape(1, EMB), beta.reshape(1, EMB), w, b.reshape(1, EMB))


def _nodeout_body(aggl_ref, aggr_ref, r_ref, pg_ref, pb_ref,
                  o1a_ref, o1b_ref, o1bias_ref, o2w_ref, o2b_ref, o_ref):
  agg = jnp.concatenate([aggl_ref[...], aggr_ref[...]], axis=-1)
  agg = _ln(agg, pg_ref[...], pb_ref[...])
  z = (jnp.dot(agg, o1a_ref[...], preferred_element_type=f32, precision=lax.Precision.HIGHEST)
       + jnp.dot(r_ref[...], o1b_ref[...], preferred_element_type=f32, precision=lax.Precision.HIGHEST)
       + o1bias_ref[...])
  z = jnp.maximum(z, 0.0)
  o_ref[...] = jnp.dot(z, o2w_ref[...], preferred_element_type=f32, precision=lax.Precision.HIGHEST) + o2b_ref[...]


def _node_out(aggl, aggr, right, pg, pb, o1_w, o1_b, o2_w, o2_b):
  n = right.shape[0]
  return pl.pallas_call(
      _nodeout_body,
      grid=(n // BN,),
      in_specs=[_row_spec(BN, HALF), _row_spec(BN, HALF), _row_spec(BN, EMB),
                _full_spec((1, EMB)), _full_spec((1, EMB)),
                _full_spec((EMB, EMB)), _full_spec((EMB, EMB)),
                _full_spec((1, EMB)), _full_spec((EMB, EMB)),
                _full_spec((1, EMB))],
      out_specs=_row_spec(BN, EMB),
      out_shape=jax.ShapeDtypeStruct((n, EMB), f32),
  )(aggl, aggr, right, pg.reshape(1, EMB), pb.reshape(1, EMB),
    o1_w[:EMB], o1_w[EMB:], o1_b.reshape(1, EMB), o2_w, o2_b.reshape(1, EMB))


def _head_body(x_ref, w1_ref, b1_ref, w2_ref, o_ref):
  h = jnp.maximum(jnp.dot(x_ref[...], w1_ref[...], preferred_element_type=f32, precision=lax.Precision.HIGHEST)
                  + b1_ref[...], 0.0)
  o_ref[...] = jnp.dot(h, w2_ref[...], preferred_element_type=f32, precision=lax.Precision.HIGHEST)


def _head(x, w1, b1, w2):
  n = x.shape[0]
  return pl.pallas_call(
      _head_body,
      grid=(n // BN,),
      in_specs=[_row_spec(BN, EMB), _full_spec((EMB, EMB)),
                _full_spec((1, EMB)), _full_spec((EMB, 1))],
      out_specs=_row_spec(BN, 1),
      out_shape=jax.ShapeDtypeStruct((n, 1), f32),
  )(x, w1, b1.reshape(1, EMB), w2)


def _sc_gather(al, ar, bl, br, dst_idx, src_idx):
  out_type = (jax.ShapeDtypeStruct((N_EDGES, HALF), f32),
              jax.ShapeDtypeStruct((N_EDGES, HALF), f32))

  @functools.partial(
      pl.kernel, out_type=out_type, mesh=_mesh,
      compiler_params=pltpu.CompilerParams(use_tc_tiling_on_sc=False),
      scratch_types=[
          pltpu.VMEM((SCH,), jnp.int32),
          pltpu.VMEM((SCH,), jnp.int32),
          pltpu.VMEM((SCH, HALF), f32),
          pltpu.VMEM((SCH, HALF), f32),
          pltpu.SemaphoreType.DMA,
          pltpu.SemaphoreType.DMA,
      ])
  def k(al_h, ar_h, bl_h, br_h, dst_h, src_h, sl_h, sr_h,
        idxd, idxs, ra, rb, sem_a, sem_b):
    sid = lax.axis_index("s")
    cid = lax.axis_index("c")

    def run(a_h, b_h, s_h):
      base0 = sid * SHARE

      def super_body(g, carry):
        base = base0 + g * SCH
        pltpu.sync_copy(dst_h.at[pl.ds(base, SCH)], idxd)
        pltpu.sync_copy(src_h.at[pl.ds(base, SCH)], idxs)
        descs = []
        for j in range(NSUB):
          sl_ = pl.ds(j * CH, CH)
          descs.append(pltpu.async_copy(a_h.at[idxd.at[sl_]], ra.at[sl_], sem_a))
          descs.append(pltpu.async_copy(b_h.at[idxs.at[sl_]], rb.at[sl_], sem_b))
        for d in descs:
          d.wait()

        def row(r, c2):
          for kk in range(HALF // 16):
            s_ = pl.ds(kk * 16, 16)
            ra[r, s_] = ra[r, s_] + rb[r, s_]
          return c2

        lax.fori_loop(0, SCH, row, 0, unroll=2)
        pltpu.sync_copy(ra, s_h.at[pl.ds(base, SCH)])
        return carry

      lax.fori_loop(0, NSUP, super_body, 0)

    @pl.when(cid == 0)
    def _():
      run(al_h, bl_h, sl_h)

    @pl.when(cid == 1)
    def _():
      run(ar_h, br_h, sr_h)

  return k(al, ar, bl, br, dst_idx, src_idx)


def _sc_scatter(ml, mr, dst2):
  out_type = (jax.ShapeDtypeStruct((N_NODES, HALF), f32),
              jax.ShapeDtypeStruct((N_NODES, HALF), f32))

  @functools.partial(
      pl.kernel, out_type=out_type, mesh=_mesh,
      compiler_params=pltpu.CompilerParams(use_tc_tiling_on_sc=False),
      scratch_types=[
          pltpu.VMEM((NSUB, CH), jnp.int32),
          pltpu.VMEM((SCH, HALF), f32),
          pltpu.VMEM((ZROWS, HALF), f32),
          pltpu.VMEM_SHARED((NPAD, HALF), f32),
      ])
  def k(ml_h, mr_h, dst2_h, aggl_h, aggr_h, idx2, rows, zbuf, acc):
    sid = lax.axis_index("s")
    cid = lax.axis_index("c")

    def zrow(r, c2):
      for kk in range(HALF // 16):
        zbuf[r, pl.ds(kk * 16, 16)] = jnp.zeros((16,), f32)
      return c2

    lax.fori_loop(0, ZROWS, zrow, 0)
    for t in range(NPAD // NS // ZROWS):
      pltpu.sync_copy(zbuf, acc.at[pl.ds(sid * (NPAD // NS) + t * ZROWS, ZROWS)])
    plsc.subcore_barrier()

    def run(m_h, agg_h):
      base0 = sid * SHARE

      def super_body(g, carry):
        base = base0 + g * SCH
        pltpu.sync_copy(dst2_h.at[pl.ds(base // CH, NSUB)], idx2)
        pltpu.sync_copy(m_h.at[pl.ds(base, SCH)], rows)
        for j in range(NSUB):
          pltpu.sync_copy(rows.at[pl.ds(j * CH, CH)], acc.at[idx2.at[j]],
                          add=True)
        return carry

      lax.fori_loop(0, NSUP, super_body, 0)
      plsc.subcore_barrier()
      pltpu.sync_copy(acc.at[pl.ds(sid * WB, WB)], agg_h.at[pl.ds(sid * WB, WB)])

    @pl.when(cid == 0)
    def _():
      run(ml_h, aggl_h)

    @pl.when(cid == 1)
    def _():
      run(mr_h, aggr_h)

  return k(ml, mr, dst2)


def _conv(cp, left, src, dst, dst2, right, b_e):
  bias_a = cp['fl_b'] + b_e * cp['fe_W'][0]
  al, ar, bl, br = _ab(right, left, cp['fl_W'], bias_a, cp['fr_W'])
  sl, sr = _sc_gather(al, ar, bl, br, dst, src)
  ml, mr = _edge_mlp(sl, sr, cp['ff_g'], cp['ff_beta'], cp['ff_W'], cp['ff_b'])
  aggl, aggr = _sc_scatter(ml, mr, dst2)
  return _node_out(aggl, aggr, right, cp['pc_g'], cp['pc_b'],
                   cp['o1_W'], cp['o1_b'], cp['o2_W'], cp['o2_b'])


@jax.jit
def kernel(constraint_features, edge_indices, edge_features, variable_features,
           params):
  p = params
  ci = edge_indices[0]
  vi = edge_indices[1]
  dst2_c = ci.reshape(N_EDGES // CH, CH)
  dst2_v = vi.reshape(N_EDGES // CH, CH)

  b_e = p['edge_ln_b'][0]

  c = _embed(constraint_features, p['cons_ln_g'], p['cons_ln_b'],
             p['cons_W1'], p['cons_b1'], p['cons_W2'], p['cons_b2'])
  v = _embed(variable_features, p['var_ln_g'], p['var_ln_b'],
             p['var_W1'], p['var_b1'], p['var_W2'], p['var_b2'])

  c = _conv(p['conv_v_to_c'], v, vi, ci, dst2_c, c, b_e)
  v = _conv(p['conv_c_to_v'], c, ci, vi, dst2_v, v, b_e)
  c = _conv(p['conv_v_to_c2'], v, vi, ci, dst2_c, c, b_e)
  v = _conv(p['conv_c_to_v2'], c, ci, vi, dst2_v, v, b_e)

  out = _head(v, p['out_W1'], p['out_b1'], p['out_W2'])
  return out[:, 0]


# speedup vs baseline: 1.6811x; 1.1632x over previous
import functools

import jax
import jax.numpy as jnp
from jax import lax
from jax.experimental import pallas as pl
from jax.experimental.pallas import tpu as pltpu
from jax.experimental.pallas import tpu_sc as plsc

EMB = 64
HALF = 32
N_NODES = 50000
N_EDGES = 800000

NC = 2
NS = 16
NW = NC * NS

GSHARE = N_EDGES // NW
GCH = 40
GNSUB = 5
GSCH = GCH * GNSUB
GNSUP = GSHARE // GSCH

SSHARE = N_EDGES // NS
SCH = 80
SNSUB = 5
SSCH = SCH * SNSUB
SNSUP = SSHARE // SSCH

NPAD = 50048
ZROWS = 391
WB = N_NODES // NS

_mesh = plsc.VectorSubcoreMesh(
    core_axis_name="c", subcore_axis_name="s", num_cores=NC, num_subcores=NS)
_sc_params = pltpu.CompilerParams(use_tc_tiling_on_sc=False)

f32 = jnp.float32
bf16 = jnp.bfloat16
HI = lax.Precision.HIGHEST


def _dot3(x, w):
  return jnp.dot(x, w, preferred_element_type=f32)


BN = 2000
BE = 8000


def _row_spec(bn, d):
  return pl.BlockSpec((bn, d), lambda i: (i, 0))


def _full_spec(shape):
  return pl.BlockSpec(shape, lambda i: tuple(0 for _ in shape))


def _ln(x, g, b):
  m = jnp.mean(x, axis=-1, keepdims=True)
  v = jnp.mean((x - m) * (x - m), axis=-1, keepdims=True)
  return (x - m) / jnp.sqrt(v + 1e-5) * g + b


def _embed_body(x_ref, w1_ref, b1_ref, w2_ref, b2_ref, o_ref):
  h = jnp.maximum(_dot3(x_ref[...], w1_ref[...]) + b1_ref[...], 0.0)
  o_ref[...] = jnp.maximum(_dot3(h, w2_ref[...]) + b2_ref[...], 0.0)


def _embed(xln, w1, b1, w2, b2):
  n, d = xln.shape
  return pl.pallas_call(
      _embed_body,
      grid=(n // BN,),
      in_specs=[_row_spec(BN, d),
                _full_spec((d, EMB)), _full_spec((1, EMB)),
                _full_spec((EMB, EMB)), _full_spec((1, EMB))],
      out_specs=_row_spec(BN, EMB),
      out_shape=jax.ShapeDtypeStruct((n, EMB), f32),
  )(xln, w1, b1.reshape(1, EMB), w2, b2.reshape(1, EMB))


def _ab_body(r_ref, l_ref, flw_ref, fla_ref, frw_ref, a_ref, b_ref):
  a_ref[...] = _dot3(r_ref[...], flw_ref[...]) + fla_ref[...]
  b_ref[...] = _dot3(l_ref[...], frw_ref[...])


def _ab(right, left, fl_w, bias_a, fr_w):
  n = right.shape[0]
  sds = jax.ShapeDtypeStruct((n, EMB), f32)
  return pl.pallas_call(
      _ab_body,
      grid=(n // BN,),
      in_specs=[_row_spec(BN, EMB), _row_spec(BN, EMB),
                _full_spec((EMB, EMB)), _full_spec((1, EMB)),
                _full_spec((EMB, EMB))],
      out_specs=[_row_spec(BN, EMB)] * 2,
      out_shape=[sds] * 2,
  )(right, left, fl_w, bias_a.reshape(1, EMB), fr_w)


def _edge_body(h_ref, w_ref, b_ref, ml_ref, mr_ref):
  msg = _dot3(h_ref[...], w_ref[...]) + b_ref[...]
  ml_ref[...] = msg[:, :HALF]
  mr_ref[...] = msg[:, HALF:]


def _edge_mlp(h, w, b):
  e = h.shape[0]
  sds = jax.ShapeDtypeStruct((e, HALF), f32)
  return pl.pallas_call(
      _edge_body,
      grid=(e // BE,),
      in_specs=[_row_spec(BE, EMB),
                _full_spec((EMB, EMB)), _full_spec((1, EMB))],
      out_specs=[_row_spec(BE, HALF)] * 2,
      out_shape=[sds] * 2,
  )(h, w, b.reshape(1, EMB))


def _nodeout_body(agg_ref, r_ref,
                  o1a_ref, o1b_ref, o1bias_ref, o2w_ref, o2b_ref, o_ref):
  agg = agg_ref[...]
  z = (_dot3(agg, o1a_ref[...])
       + _dot3(r_ref[...], o1b_ref[...])
       + o1bias_ref[...])
  z = jnp.maximum(z, 0.0)
  o_ref[...] = _dot3(z, o2w_ref[...]) + o2b_ref[...]


def _node_out(lnagg, right, o1_w, o1_b, o2_w, o2_b):
  n = right.shape[0]
  return pl.pallas_call(
      _nodeout_body,
      grid=(n // BN,),
      in_specs=[_row_spec(BN, EMB), _row_spec(BN, EMB),
                _full_spec((EMB, EMB)), _full_spec((EMB, EMB)),
                _full_spec((1, EMB)), _full_spec((EMB, EMB)),
                _full_spec((1, EMB))],
      out_specs=_row_spec(BN, EMB),
      out_shape=jax.ShapeDtypeStruct((n, EMB), f32),
  )(lnagg, right,
    o1_w[:EMB], o1_w[EMB:], o1_b.reshape(1, EMB), o2_w, o2_b.reshape(1, EMB))


def _head_body(x_ref, w1_ref, b1_ref, w2_ref, o_ref):
  h = jnp.maximum(_dot3(x_ref[...], w1_ref[...]) + b1_ref[...], 0.0)
  o_ref[...] = _dot3(h, w2_ref[...])


def _head(x, w1, b1, w2):
  n = x.shape[0]
  return pl.pallas_call(
      _head_body,
      grid=(n // BN,),
      in_specs=[_row_spec(BN, EMB), _full_spec((EMB, EMB)),
                _full_spec((1, EMB)), _full_spec((EMB, 1))],
      out_specs=_row_spec(BN, 1),
      out_shape=jax.ShapeDtypeStruct((n, 1), f32),
  )(x, w1, b1.reshape(1, EMB), w2)


def _sc_gather(a, b, dst_idx, src_idx):
  out_type = jax.ShapeDtypeStruct((N_EDGES, EMB), f32)

  @functools.partial(
      pl.kernel, out_type=out_type, mesh=_mesh, compiler_params=_sc_params,
      scratch_types=[
          pltpu.VMEM((GSCH,), jnp.int32), pltpu.VMEM((GSCH,), jnp.int32),
          pltpu.VMEM((GSCH,), jnp.int32), pltpu.VMEM((GSCH,), jnp.int32),
          pltpu.VMEM((GSCH, EMB), f32), pltpu.VMEM((GSCH, EMB), f32),
          pltpu.VMEM((GSCH, EMB), f32), pltpu.VMEM((GSCH, EMB), f32),
          pltpu.SemaphoreType.DMA, pltpu.SemaphoreType.DMA,
          pltpu.SemaphoreType.DMA, pltpu.SemaphoreType.DMA,
          pltpu.SemaphoreType.DMA, pltpu.SemaphoreType.DMA,
          pltpu.SemaphoreType.DMA, pltpu.SemaphoreType.DMA,
      ])
  def k(a_h, b_h, dst_h, src_h, s_h,
        idxd0, idxd1, idxs0, idxs1, ra0, ra1, rb0, rb1,
        semi0, semi1, sema0, sema1, semb0, semb1, semw0, semw1):
    sid = lax.axis_index("s")
    cid = lax.axis_index("c")
    wid = sid * NC + cid
    base0 = wid * GSHARE
    idxd = (idxd0, idxd1)
    idxs = (idxs0, idxs1)
    ra = (ra0, ra1)
    rb = (rb0, rb1)
    semi = (semi0, semi1)
    sema = (sema0, sema1)
    semb = (semb0, semb1)
    semw = (semw0, semw1)

    def idx_issue(g, p):
      base = base0 + g * GSCH
      pltpu.async_copy(dst_h.at[pl.ds(base, GSCH)], idxd[p], semi[p])
      pltpu.async_copy(src_h.at[pl.ds(base, GSCH)], idxs[p], semi[p])

    def idx_wait(p):
      pltpu.make_async_copy(dst_h.at[pl.ds(0, GSCH)], idxd[p], semi[p]).wait()
      pltpu.make_async_copy(dst_h.at[pl.ds(0, GSCH)], idxs[p], semi[p]).wait()

    def gather_issue(p):
      for j in range(GNSUB):
        sl_ = pl.ds(j * GCH, GCH)
        pltpu.async_copy(a_h.at[idxd[p].at[sl_]], ra[p].at[sl_], sema[p])
        pltpu.async_copy(b_h.at[idxs[p].at[sl_]], rb[p].at[sl_], semb[p])

    def gather_wait(p):
      pltpu.make_async_copy(s_h.at[pl.ds(0, GSCH)], ra[p], sema[p]).wait()
      pltpu.make_async_copy(s_h.at[pl.ds(0, GSCH)], rb[p], semb[p]).wait()

    def wb_issue(g, p):
      base = base0 + g * GSCH
      pltpu.async_copy(ra[p], s_h.at[pl.ds(base, GSCH)], semw[p])

    def wb_wait(p):
      pltpu.make_async_copy(ra[p], s_h.at[pl.ds(0, GSCH)], semw[p]).wait()

    def compute(p):
      def row(r, c2):
        for kk in range(EMB // 16):
          s_ = pl.ds(kk * 16, 16)
          ra[p][r, s_] = ra[p][r, s_] + rb[p][r, s_]
        return c2
      lax.fori_loop(0, GSCH, row, 0, unroll=2)

    def chunk(g, p):
      gather_wait(p)

      @pl.when(g + 2 < GNSUP)
      def _():
        idx_issue(g + 2, p)

      @pl.when(g + 1 < GNSUP)
      def _():
        idx_wait(1 - p)

        @pl.when(g >= 1)
        def _():
          wb_wait(1 - p)

        gather_issue(1 - p)

      compute(p)
      wb_issue(g, p)

    idx_issue(0, 0)
    idx_issue(1, 1)
    idx_wait(0)
    gather_issue(0)

    def pair(i, c2):
      chunk(2 * i, 0)
      chunk(2 * i + 1, 1)
      return c2

    lax.fori_loop(0, GNSUP // 2, pair, 0)
    chunk(GNSUP - 1, (GNSUP - 1) % 2)
    wb_wait(0)
    wb_wait(1)

  return k(a, b, dst_idx, src_idx)


def _sc_scatter(ml, mr, dst2):
  sds = jax.ShapeDtypeStruct((N_NODES, HALF), f32)
  out_type = (sds, sds)

  @functools.partial(
      pl.kernel, out_type=out_type, mesh=_mesh, compiler_params=_sc_params,
      scratch_types=[
          pltpu.VMEM((SNSUB, SCH), jnp.int32),
          pltpu.VMEM((SNSUB, SCH), jnp.int32),
          pltpu.VMEM((SSCH, HALF), f32), pltpu.VMEM((SSCH, HALF), f32),
          pltpu.VMEM_SHARED((NPAD, HALF), f32),
          pltpu.SemaphoreType.DMA, pltpu.SemaphoreType.DMA,
          pltpu.SemaphoreType.DMA, pltpu.SemaphoreType.DMA,
      ])
  def k(ml_h, mr_h, dst2_h, aggl_h, aggr_h, idx0, idx1, rows0, rows1,
        acc, seml0, seml1, sems0, sems1):
    sid = lax.axis_index("s")
    cid = lax.axis_index("c")
    base0 = sid * SSHARE
    idx = (idx0, idx1)
    rows = (rows0, rows1)
    seml = (seml0, seml1)
    sems = (sems0, sems1)

    def zrow(r, c2):
      for kk in range(HALF // 16):
        rows0[r, pl.ds(kk * 16, 16)] = jnp.zeros((16,), f32)
      return c2

    lax.fori_loop(0, SSCH, zrow, 0)
    zb = NPAD // NS
    for t in range(zb // SSCH):
      pltpu.sync_copy(rows0, acc.at[pl.ds(sid * zb + t * SSCH, SSCH)])
    rem = zb % SSCH
    pltpu.sync_copy(rows0.at[pl.ds(0, rem)],
                    acc.at[pl.ds(sid * zb + (zb // SSCH) * SSCH, rem)])
    plsc.subcore_barrier()

    def run(m_h, agg_h):
      def load_issue(g, p):
        base = base0 + g * SSCH
        pltpu.async_copy(dst2_h.at[pl.ds(base // SCH, SNSUB)], idx[p], seml[p])
        pltpu.async_copy(m_h.at[pl.ds(base, SSCH)], rows[p], seml[p])

      def load_wait(p):
        pltpu.make_async_copy(dst2_h.at[pl.ds(0, SNSUB)], idx[p],
                              seml[p]).wait()
        pltpu.make_async_copy(m_h.at[pl.ds(0, SSCH)], rows[p], seml[p]).wait()

      def scat_issue(p):
        for j in range(SNSUB):
          pltpu.async_copy(rows[p].at[pl.ds(j * SCH, SCH)],
                           acc.at[idx[p].at[j]], sems[p], add=True)

      def scat_wait(p):
        pltpu.make_async_copy(m_h.at[pl.ds(0, SSCH)], rows[p], sems[p]).wait()

      def chunk(g, p):
        load_wait(p)

        @pl.when(g >= 1)
        def _():
          scat_wait(1 - p)

        @pl.when(g + 1 < SNSUP)
        def _():
          load_issue(g + 1, 1 - p)

        scat_issue(p)

      load_issue(0, 0)

      def pair(i, c2):
        chunk(2 * i, 0)
        chunk(2 * i + 1, 1)
        return c2

      lax.fori_loop(0, SNSUP // 2, pair, 0)
      chunk(SNSUP - 1, (SNSUP - 1) % 2)
      scat_wait((SNSUP - 1) % 2)

      plsc.subcore_barrier()
      pltpu.sync_copy(acc.at[pl.ds(sid * WB, WB)],
                      agg_h.at[pl.ds(sid * WB, WB)])

    @pl.when(cid == 0)
    def _():
      run(ml_h, aggl_h)

    @pl.when(cid == 1)
    def _():
      run(mr_h, aggr_h)

  return k(ml, mr, dst2)


def _conv(cp, left, src, dst, dst2, right, b_e):
  bias_a = cp['fl_b'] + b_e * cp['fe_W'][0]
  a, b = _ab(right, left, cp['fl_W'], bias_a, cp['fr_W'])
  s = _sc_gather(a, b, dst, src)
  h = jnp.maximum(_ln(s, cp['ff_g'], cp['ff_beta']), 0.0)
  ml, mr = _edge_mlp(h, cp['ff_W'], cp['ff_b'])
  aggl, aggr = _sc_scatter(ml, mr, dst2)
  lnagg = _ln(jnp.concatenate([aggl, aggr], axis=-1), cp['pc_g'], cp['pc_b'])
  return _node_out(lnagg, right, cp['o1_W'], cp['o1_b'], cp['o2_W'],
                   cp['o2_b'])


@jax.jit
def kernel(constraint_features, edge_indices, edge_features, variable_features,
           params):
  p = params
  ci = edge_indices[0]
  vi = edge_indices[1]
  dst2_c = ci.reshape(N_EDGES // SCH, SCH)
  dst2_v = vi.reshape(N_EDGES // SCH, SCH)

  b_e = p['edge_ln_b'][0]

  cln = _ln(constraint_features, p['cons_ln_g'], p['cons_ln_b'])
  vln = _ln(variable_features, p['var_ln_g'], p['var_ln_b'])
  c = _embed(cln, p['cons_W1'], p['cons_b1'], p['cons_W2'], p['cons_b2'])
  v = _embed(vln, p['var_W1'], p['var_b1'], p['var_W2'], p['var_b2'])

  c = _conv(p['conv_v_to_c'], v, vi, ci, dst2_c, c, b_e)
  v = _conv(p['conv_c_to_v'], c, ci, vi, dst2_v, v, b_e)
  c = _conv(p['conv_v_to_c2'], v, vi, ci, dst2_c, c, b_e)
  v = _conv(p['conv_c_to_v2'], c, ci, vi, dst2_v, v, b_e)

  out = _head(v, p['out_W1'], p['out_b1'], p['out_W2'])
  return out[:, 0]
